# baseline (speedup 1.0000x reference)
import functools

import jax
import jax.numpy as jnp
from jax import lax
from jax.experimental import pallas as pl
from jax.experimental.pallas import tpu as pltpu

TOP_K = 8
N_EXPERTS = 64
IN_FEATURES = 1024
OUT_FEATURES = 64
N_TOKENS = 2048

BLK_N = 512
_NBLK = N_TOKENS // BLK_N


def _body(f_ref, a_ref, w_ref, bflat_ref, out_ref, acc_a, acc_b):
    s = pl.program_id(0)

    def produce(buf):
        f = f_ref[...].astype(jnp.bfloat16)
        buf[...] = jnp.dot(f, w_ref[...],
                           preferred_element_type=jnp.float32)

    def consume(buf):
        acc = buf[...].astype(jnp.bfloat16) + bflat_ref[...]

        a = a_ref[...]
        lane = lax.broadcasted_iota(jnp.int32, (BLK_N, N_EXPERTS), 1)
        sel = []
        for _ in range(TOP_K):
            m = jnp.min(a, axis=1, keepdims=True)
            cand = jnp.where(a == m, lane, N_EXPERTS)
            amin = jnp.min(cand, axis=1, keepdims=True)
            sel.append(amin)
            a = jnp.where(lane == amin, jnp.inf, a)

        for k in range(TOP_K):
            e = sel[k]
            cur = acc
            width = (N_EXPERTS // 2) * OUT_FEATURES
            for bit in range(5, -1, -1):
                take_hi = ((e >> bit) & 1) == 1
                cur = jnp.where(take_hi, cur[:, width:], cur[:, :width])
                width //= 2
            out_ref[:, k * OUT_FEATURES:(k + 1) * OUT_FEATURES] = (
                cur.astype(jnp.float32))

    @pl.when(s % 2 == 0)
    def _even():
        produce(acc_a)
        consume(acc_b)

    @pl.when(s % 2 == 1)
    def _odd():
        produce(acc_b)
        consume(acc_a)


@jax.jit
def kernel(features, activated, W, b):
    wr = W.transpose(1, 0, 2).reshape(IN_FEATURES, N_EXPERTS * OUT_FEATURES)
    wr = wr.astype(jnp.bfloat16)
    bflat = b.reshape(1, N_EXPERTS * OUT_FEATURES).astype(jnp.bfloat16)

    out = pl.pallas_call(
        _body,
        grid=(_NBLK + 1,),
        in_specs=[
            pl.BlockSpec((BLK_N, IN_FEATURES),
                         lambda s: (jnp.minimum(s, _NBLK - 1), 0)),
            pl.BlockSpec((BLK_N, N_EXPERTS),
                         lambda s: (jnp.maximum(s - 1, 0), 0)),
            pl.BlockSpec((IN_FEATURES, N_EXPERTS * OUT_FEATURES),
                         lambda s: (0, 0)),
            pl.BlockSpec((1, N_EXPERTS * OUT_FEATURES), lambda s: (0, 0)),
        ],
        out_specs=pl.BlockSpec((BLK_N, TOP_K * OUT_FEATURES),
                               lambda s: (jnp.maximum(s - 1, 0), 0)),
        out_shape=jax.ShapeDtypeStruct(
            (N_TOKENS, TOP_K * OUT_FEATURES), jnp.float32),
        scratch_shapes=[
            pltpu.VMEM((BLK_N, N_EXPERTS * OUT_FEATURES), jnp.float32),
            pltpu.VMEM((BLK_N, N_EXPERTS * OUT_FEATURES), jnp.float32),
        ],
    )(features, activated, wr, bflat)
    return out.reshape(N_TOKENS, TOP_K, OUT_FEATURES)
